# baseline (speedup 1.0000x reference)
import functools

import jax
import jax.numpy as jnp
from jax import lax
from jax.experimental import pallas as pl
from jax.experimental.pallas import tpu as pltpu
from jax.experimental.pallas import tpu_sc as plsc

_NBINS = (64, 32)
_NG = 5000
_NR = 16
_NRXG = _NR * _NG
_NPAD = 524288
_B = 2048
_CH0 = 16384
_CH1 = 16384
_F32 = jnp.float32
_I32 = jnp.int32


def _shift_cumsum(x, n):
    s = 1
    while s < n:
        pad = jnp.zeros(x.shape[:-1] + (s,), x.dtype)
        x = x + jnp.concatenate([pad, x[..., :-s]], -1)
        s *= 2
    return x


def _prep_kernel(uh_ref, uw_ref, dh_ref,
                 h0_ref, cf0_ref, gm0_ref, cw0_ref,
                 h1_ref, cf1_ref, gm1_ref, cw1_ref):
    uh = uh_ref[...]
    uw = uw_ref[...]
    dh = dh_ref[0]
    zcol = jnp.zeros((_NG, 1), _F32)
    uw0 = uw[:, :63]
    ew0 = jnp.exp(uw0 - jnp.max(uw0, axis=-1, keepdims=True))
    w0 = ew0 / jnp.sum(ew0, axis=-1, keepdims=True)
    cw0_ref[...] = _shift_cumsum(jnp.concatenate([zcol, w0], -1), 64)
    e0 = jnp.exp(uh[:, :64] + dh[:, :64])
    tz0 = (e0[:, :-1] + e0[:, 1:]) * _F32(0.5) * w0
    pg0 = jnp.sum(tz0, axis=-1, keepdims=True) * _F32(1.0 / _NG)
    area0 = jnp.sum(pg0)
    h0_ref[0] = e0 / area0
    gm0 = pg0 / area0
    gm0_ref[0] = gm0
    cf0_ref[0] = _shift_cumsum(
        jnp.concatenate([zcol, tz0 / area0], -1), 64) * _F32(1.0 / _NG)
    uw1 = uw[:, 63:]
    ew1 = jnp.exp(uw1 - jnp.max(uw1, axis=-1, keepdims=True))
    w1 = ew1 / jnp.sum(ew1, axis=-1, keepdims=True)
    cw1_ref[...] = _shift_cumsum(jnp.concatenate([zcol, w1], -1), 32)
    e1 = jnp.exp(uh[:, 64:] + dh[:, 64:])
    tz1 = (e1[:, :-1] + e1[:, 1:]) * _F32(0.5) * w1
    pg1 = jnp.sum(tz1, axis=-1, keepdims=True) * gm0
    area1 = jnp.sum(pg1)
    h1_ref[0] = e1 / area1
    gm1_ref[0] = pg1 / area1
    cf1_ref[0] = _shift_cumsum(
        jnp.concatenate([zcol, tz1 / area1], -1), 32) * gm0


def _bl_kernel(c0_ref, cf0_ref, c1_ref, cf1_ref, bl0_ref, bl1_ref):
    bl0_ref[0] = c0_ref[0] + cf0_ref[0]
    bl1_ref[0] = c1_ref[0] + cf1_ref[0]


def _tables(dh_all, uh_all, uw_all):
    n0, n1 = _NBINS
    sds = jax.ShapeDtypeStruct
    full2 = lambda shp: pl.BlockSpec(shp, lambda r: (0, 0))
    per_r = lambda shp: pl.BlockSpec((1,) + shp, lambda r: (r, 0, 0))
    h0, cf0, gm0, cw0, h1, cf1, gm1, cw1 = pl.pallas_call(
        _prep_kernel,
        grid=(_NR,),
        in_specs=[full2((_NG, 96)), full2((_NG, 94)), per_r((_NG, 96))],
        out_specs=[per_r((_NG, n0)), per_r((_NG, n0)), per_r((_NG, 1)),
                   full2((_NG, n0)),
                   per_r((_NG, n1)), per_r((_NG, n1)), per_r((_NG, 1)),
                   full2((_NG, n1))],
        out_shape=[sds((_NR, _NG, n0), _F32), sds((_NR, _NG, n0), _F32),
                   sds((_NR, _NG, 1), _F32), sds((_NG, n0), _F32),
                   sds((_NR, _NG, n1), _F32), sds((_NR, _NG, n1), _F32),
                   sds((_NR, _NG, 1), _F32), sds((_NG, n1), _F32)],
    )(uh_all, uw_all, dh_all)
    gm0f = gm0.reshape(_NR, _NG)
    gm1f = gm1.reshape(_NR, _NG)
    C0 = jnp.pad(jnp.cumsum(gm0f, -1), ((0, 0), (1, 0)))[:, :-1]
    C1 = jnp.pad(jnp.cumsum(gm1f, -1), ((0, 0), (1, 0)))[:, :-1]
    bl0, bl1 = pl.pallas_call(
        _bl_kernel,
        grid=(_NR,),
        in_specs=[per_r((_NG, 1)), per_r((_NG, n0)),
                  per_r((_NG, 1)), per_r((_NG, n1))],
        out_specs=[per_r((_NG, n0)), per_r((_NG, n1))],
        out_shape=[sds((_NR, _NG, n0), _F32), sds((_NR, _NG, n1), _F32)],
    )(C0.reshape(_NR, _NG, 1), cf0, C1.reshape(_NR, _NG, 1), cf1)
    l0 = (cw0.reshape(-1), h0.reshape(-1), bl0.reshape(-1))
    l1 = (cw1.reshape(-1), h1.reshape(-1), bl1.reshape(-1))
    return (l0, l1), (C0.reshape(-1), gm0f.reshape(-1))


def _iota16():
    return lax.broadcasted_iota(_I32, (16,), 0)


def _sc_eval(xp, rxgp, gp, tabs):
    cumw0, h0, bl0, cumw1, h1, bl1, gl2t, gw2t = tabs
    info = plsc.get_sparse_core_info()
    ns = info.num_subcores
    nv = _B // 16
    mesh = plsc.VectorSubcoreMesh(core_axis_name="c", subcore_axis_name="s")

    @functools.partial(
        pl.kernel,
        mesh=mesh,
        out_type=[jax.ShapeDtypeStruct((_NPAD,), _F32),
                  jax.ShapeDtypeStruct((_NPAD,), _F32)],
        scratch_types=[
            pltpu.VMEM((_B,), _F32),
            pltpu.VMEM((_B,), _I32),
            pltpu.VMEM((_B,), _I32),
            pltpu.VMEM((_B,), _F32),
            pltpu.VMEM((_B,), _I32),
            pltpu.VMEM((_B,), _I32),
            pltpu.VMEM((_B,), _I32),
            pltpu.VMEM((_B,), _F32),
            pltpu.VMEM((_B,), _F32),
            pltpu.VMEM((_B,), _F32),
            pltpu.VMEM((_B,), _F32),
            pltpu.VMEM((_B,), _F32),
            pltpu.VMEM((_B,), _F32),
            pltpu.VMEM((_B,), _F32),
            pltpu.VMEM((_B,), _F32),
            pltpu.SemaphoreType.DMA,
        ],
    )
    def k(x_hbm, rxg_hbm, g_hbm, cumw0_hbm, h0_hbm, bl0_hbm,
          cumw1_hbm, h1_hbm, bl1_hbm, gl2_hbm, gw2_hbm,
          out_hbm, dp_hbm,
          x_v, rxg_v, g_v, t_v, c_v, ia_v, ib_v, val_v, cwk_v, h_v, hn_v,
          bl_v, gl_v, gw_v, dp_v, sem):
        cidx = lax.axis_index("c")
        sidx = lax.axis_index("s")
        is0 = cidx == 0
        tile_base = jnp.where(is0, sidx * _CH0, ns * _CH0 + sidx * _CH1)
        nsub = jnp.where(is0, _CH0 // _B, _CH1 // _B)

        def vloop(body):
            lax.fori_loop(0, nv, lambda i, _: (body(i, pl.ds(i * 16, 16)), 0)[1], 0)

        def rnd(tab, idx_v, dst_v):
            pltpu.async_copy(tab.at[idx_v], dst_v, sem).wait()

        def layer(li, n, cumw_t, h_t, bl_t):
            bits = [32, 16, 8, 4, 2, 1] if n == 64 else [16, 8, 4, 2, 1]

            def init(i, sl):
                x = x_v[sl]
                if li == 0:
                    gl = g_v[sl].astype(_F32) / _F32(_NG)
                    gw = jnp.full((16,), 1.0 / _NG, _F32)
                else:
                    gl = gl_v[sl]
                    gw = gw_v[sl]
                t = (x - gl) / gw
                t_v[sl] = t
                c_v[sl] = jnp.zeros((16,), _I32)
                ia_v[sl] = g_v[sl] * n + (bits[0] - 1)

            vloop(init)
            rnd(cumw_t, ia_v, val_v)
            for bi in range(1, len(bits)):
                b_prev, b = bits[bi - 1], bits[bi]

                def step(i, sl, b_prev=b_prev, b=b):
                    c = c_v[sl] + jnp.where(
                        val_v[sl] < t_v[sl],
                        jnp.full((16,), b_prev, _I32), jnp.zeros((16,), _I32))
                    c_v[sl] = c
                    ia_v[sl] = g_v[sl] * n + (c + (b - 1))

                vloop(step)
                rnd(cumw_t, ia_v, val_v)

            def fin(i, sl):
                c = c_v[sl] + jnp.where(
                    val_v[sl] < t_v[sl],
                    jnp.ones((16,), _I32), jnp.zeros((16,), _I32))
                kk = jnp.clip(c - 1, 0, n - 2)
                c_v[sl] = kk
                ia_v[sl] = g_v[sl] * n + kk
                ib_v[sl] = rxg_v[sl] * n + kk

            vloop(fin)
            rnd(cumw_t, ia_v, cwk_v)
            rnd(h_t, ib_v, h_v)
            rnd(bl_t, ib_v, bl_v)

            def bump(i, sl):
                ia_v[sl] = ia_v[sl] + jnp.ones((16,), _I32)
                ib_v[sl] = ib_v[sl] + jnp.ones((16,), _I32)

            vloop(bump)
            rnd(cumw_t, ia_v, val_v)
            rnd(h_t, ib_v, hn_v)

            def apply(i, sl):
                x = x_v[sl]
                if li == 0:
                    gl = g_v[sl].astype(_F32) / _F32(_NG)
                    gw = jnp.full((16,), 1.0 / _NG, _F32)
                else:
                    gl = gl_v[sl]
                    gw = gw_v[sl]
                cwk = cwk_v[sl]
                bw = gw * (val_v[sl] - cwk)
                left = gl + gw * cwk
                a = jnp.clip((x - left) / jnp.maximum(bw, _F32(1e-12)),
                             0.0, 1.0)
                hl = h_v[sl]
                dh = hn_v[sl] - hl
                x_v[sl] = bl_v[sl] + a * bw * hl + _F32(0.5) * a * a * bw * dh
                f = jnp.maximum(hl + a * dh, _F32(1e-12))
                if li == 0:
                    dp_v[sl] = f
                else:
                    dp_v[sl] = dp_v[sl] * f

            vloop(apply)

        def sub(s, _):
            base = tile_base + s * _B
            pltpu.sync_copy(x_hbm.at[pl.ds(base, _B)], x_v)
            pltpu.sync_copy(rxg_hbm.at[pl.ds(base, _B)], rxg_v)
            pltpu.sync_copy(g_hbm.at[pl.ds(base, _B)], g_v)
            layer(0, 64, cumw0_hbm, h0_hbm, bl0_hbm)
            rnd(gl2_hbm, rxg_v, gl_v)
            rnd(gw2_hbm, rxg_v, gw_v)
            layer(1, 32, cumw1_hbm, h1_hbm, bl1_hbm)
            pltpu.sync_copy(x_v, out_hbm.at[pl.ds(base, _B)])
            pltpu.sync_copy(dp_v, dp_hbm.at[pl.ds(base, _B)])
            return 0

        lax.fori_loop(0, nsub, sub, 0)

    return k(xp, rxgp, gp, cumw0, h0, bl0, cumw1, h1, bl1, gl2t, gw2t)


def _log_kernel(dp_ref, o_ref):
    o_ref[...] = jnp.log(dp_ref[...])


def kernel(cut_positions, cut_local_reflatentxgene_ix, cut_local_gene_ix,
           cut_local_reflatent_ix, mixture_delta_reflatentxgene,
           unnormalized_heights, unnormalized_widths):
    del cut_local_reflatent_ix
    n = cut_positions.shape[0]
    (l0, l1), (gl2t, gw2t) = _tables(mixture_delta_reflatentxgene,
                                     unnormalized_heights, unnormalized_widths)
    pad = _NPAD - n
    xp = jnp.pad(cut_positions, (0, pad))
    rxgp = jnp.pad(cut_local_reflatentxgene_ix.astype(_I32), (0, pad))
    gp = jnp.pad(cut_local_gene_ix.astype(_I32), (0, pad))
    out_p, dp_p = _sc_eval(xp, rxgp, gp, (*l0, *l1, gl2t, gw2t))
    lad_p = pl.pallas_call(
        _log_kernel,
        out_shape=jax.ShapeDtypeStruct((_NPAD // 128, 128), _F32),
    )(dp_p.reshape(_NPAD // 128, 128)).reshape(-1)
    return out_p[:n], lad_p[:n]
